# baseline (speedup 1.0000x reference)
import functools

import jax
import jax.numpy as jnp
from jax import lax
from jax.experimental import pallas as pl
from jax.experimental.pallas import tpu as pltpu
from jax.experimental.pallas import tpu_sc as plsc

_NC = 2
_NS = 16
_NW = _NC * _NS
_CAP = 512
_G = 2000
_K = 128
_FLAT = 11520
_SHIFT = 14


def _rows_per_tile(n):
    return (-(-n // _NW) + 7) // 8 * 8


def _make_binner(n, e):
    tpe = e // _NW
    rng = _rows_per_tile(n)
    mesh = plsc.VectorSubcoreMesh(core_axis_name="c", subcore_axis_name="s")

    @functools.partial(
        pl.kernel,
        out_type=(
            jax.ShapeDtypeStruct((_NW, _NW, _CAP), jnp.int32),
            jax.ShapeDtypeStruct((_NW, _NW, 128), jnp.int32),
        ),
        mesh=mesh,
        compiler_params=pltpu.CompilerParams(needs_layout_passes=False),
        scratch_types=[
            pltpu.VMEM((_G,), jnp.int32),
            pltpu.VMEM((_G,), jnp.int32),
            pltpu.VMEM((_NW * _CAP,), jnp.int32),
            pltpu.VMEM((_NW, 128), jnp.int32),
        ],
    )
    def binner(src_hbm, dst_hbm, buck_hbm, cnt_hbm, srcb, dstb, buck, cntv):
        c = lax.axis_index("c")
        s = lax.axis_index("s")
        w = c * _NS + s
        ebase = w * tpe
        for r in range(_NW):
            cntv[r, pl.ds(0, 16)] = jnp.zeros((16,), jnp.int32)

        def _chunk(ch, carry):
            pltpu.sync_copy(src_hbm.at[pl.ds(ebase + ch * _G, _G)], srcb)
            pltpu.sync_copy(dst_hbm.at[pl.ds(ebase + ch * _G, _G)], dstb)

            def _group(g, carry):
                sv = srcb[pl.ds(g * 16, 16)]
                dv = dstb[pl.ds(g * 16, 16)]
                pk = sv * (1 << _SHIFT) + dv
                for r in range(_NW):
                    m = (dv >= r * rng) & (dv < (r + 1) * rng)
                    cvec = cntv[r, pl.ds(0, 16)]
                    pos = cvec + plsc.cumsum(m.astype(jnp.int32)) - 1
                    plsc.store_scatter(buck, [pos + r * _CAP], pk, mask=m)
                    cntv[r, pl.ds(0, 16)] = (
                        cvec + plsc.all_reduce_population_count(m))
                return carry

            return lax.fori_loop(0, _G // 16, _group, carry)

        lax.fori_loop(0, tpe // _G, _chunk, 0)
        for r in range(_NW):
            pltpu.sync_copy(buck.at[pl.ds(r * _CAP, _CAP)], buck_hbm.at[r, w])
            pltpu.sync_copy(cntv.at[pl.ds(r, 1)], cnt_hbm.at[r, pl.ds(w, 1)])

    return binner


def _make_segsum(n, e, d):
    rng = _rows_per_tile(n)
    srows = rng + 16
    npad = _NW * rng
    mesh = plsc.VectorSubcoreMesh(core_axis_name="c", subcore_axis_name="s")

    @functools.partial(
        pl.kernel,
        out_type=jax.ShapeDtypeStruct((npad, d), jnp.float32),
        mesh=mesh,
        compiler_params=pltpu.CompilerParams(needs_layout_passes=False),
        scratch_types=[
            pltpu.VMEM((_NW, _CAP), jnp.int32),
            pltpu.VMEM((_NW, 128), jnp.int32),
            pltpu.VMEM((_FLAT,), jnp.int32),
            pltpu.VMEM((_K,), jnp.int32),
            pltpu.VMEM((_K,), jnp.int32),
            pltpu.VMEM((_K,), jnp.int32),
            pltpu.VMEM((_K,), jnp.int32),
            pltpu.VMEM((_K, d), jnp.float32),
            pltpu.VMEM((_K, d), jnp.float32),
            pltpu.VMEM((16, d), jnp.float32),
            pltpu.VMEM_SHARED((_NS * srows, d), jnp.float32),
            pltpu.SemaphoreType.DMA,
            pltpu.SemaphoreType.DMA,
        ],
    )
    def segsum(h_hbm, buck_hbm, cnt_hbm, out_hbm, buckv, cntv, flat, srca,
               dsta, srcb, dstb, rowsa, rowsb, zbuf, acc, sema, semb):
        c = lax.axis_index("c")
        s = lax.axis_index("s")
        w = c * _NS + s
        sbase = s * srows

        def _zero_row(i, carry):
            for jj in range(d // 16):
                zbuf[i, pl.ds(jj * 16, 16)] = jnp.zeros((16,), jnp.float32)
            return carry

        lax.fori_loop(0, 16, _zero_row, 0)

        def _zero_acc(i, carry):
            pltpu.sync_copy(zbuf, acc.at[pl.ds(sbase + i * 16, 16)])
            return carry

        lax.fori_loop(0, srows // 16, _zero_acc, 0)

        pltpu.sync_copy(buck_hbm.at[w], buckv)
        pltpu.sync_copy(cnt_hbm.at[w], cntv)

        padv = jnp.full((16,), w * rng + rng, jnp.int32)
        lane = lax.iota(jnp.int32, 16)

        def _prefill(i, carry):
            flat[pl.ds(i * 16, 16)] = padv
            return carry

        lax.fori_loop(0, _FLAT // 16, _prefill, 0)

        tot = jnp.int32(0)
        for s2 in range(_NW):
            cnt = cntv[s2, pl.ds(0, 16)][0]

            def _cp(k, carry, s2=s2, cnt=cnt, tot=tot):
                vals = buckv[s2, pl.ds(k * 16, 16)]
                m = (k * 16 + lane) < cnt
                flat[pl.ds(tot + k * 16, 16)] = jnp.where(m, vals, padv)
                return carry

            lax.fori_loop(0, (cnt + 15) // 16, _cp, 0)
            tot = tot + cnt

        off = sbase - w * rng
        nch = (tot + _K - 1) // _K
        npairs = (nch + 1) // 2

        def _unpack(ch, srcc, dstc):
            for k in range(_K // 16):
                p = flat[pl.ds(ch * _K + k * 16, 16)]
                srcc[pl.ds(k * 16, 16)] = p >> _SHIFT
                dstc[pl.ds(k * 16, 16)] = (p & ((1 << _SHIFT) - 1)) + off

        _unpack(0, srca, dsta)
        pltpu.async_copy(h_hbm.at[srca], rowsa, sema)

        def _pair(j2, carry):
            ch = j2 * 2
            _unpack(ch + 1, srcb, dstb)
            pltpu.async_copy(h_hbm.at[srcb], rowsb, semb)
            pltpu.make_async_copy(h_hbm.at[srca], rowsa, sema).wait()
            pltpu.sync_copy(rowsa, acc.at[dsta], add=True)
            _unpack(ch + 2, srca, dsta)
            pltpu.async_copy(h_hbm.at[srca], rowsa, sema)
            pltpu.make_async_copy(h_hbm.at[srcb], rowsb, semb).wait()
            pltpu.sync_copy(rowsb, acc.at[dstb], add=True)
            return carry

        lax.fori_loop(0, npairs, _pair, 0)
        pltpu.make_async_copy(h_hbm.at[srca], rowsa, sema).wait()

        pltpu.sync_copy(acc.at[pl.ds(sbase, rng)],
                        out_hbm.at[pl.ds(w * rng, rng)])

    return segsum


def _make_mlp(n, d, blk, final_relu):
    def body(x_ref, p_ref, wa_ref, ba_ref, wb_ref, bb_ref, o_ref):
        z = x_ref[...] + p_ref[...]
        t = jnp.dot(z, wa_ref[...], preferred_element_type=jnp.float32)
        t = jnp.maximum(t + ba_ref[...], 0.0)
        y = jnp.dot(t, wb_ref[...], preferred_element_type=jnp.float32)
        y = y + bb_ref[...]
        if final_relu:
            y = jnp.maximum(y, 0.0)
        o_ref[...] = y

    return pl.pallas_call(
        body,
        grid=(n // blk,),
        in_specs=[
            pl.BlockSpec((blk, d), lambda i: (i, 0)),
            pl.BlockSpec((blk, d), lambda i: (i, 0)),
            pl.BlockSpec((d, d), lambda i: (0, 0)),
            pl.BlockSpec((1, d), lambda i: (0, 0)),
            pl.BlockSpec((d, d), lambda i: (0, 0)),
            pl.BlockSpec((1, d), lambda i: (0, 0)),
        ],
        out_specs=pl.BlockSpec((blk, d), lambda i: (i, 0)),
        out_shape=jax.ShapeDtypeStruct((n, d), jnp.float32),
    )


def kernel(x, edge_index, W1a, b1a, W1b, b1b, W2a, b2a, W2b, b2b):
    n, d = x.shape
    e = edge_index.shape[1]
    src = edge_index[0]
    dst = edge_index[1]

    binner = _make_binner(n, e)
    segsum = _make_segsum(n, e, d)
    mlp1 = _make_mlp(n, d, 2000, True)
    mlp2 = _make_mlp(n, d, 2000, False)

    buckets, counts = binner(src, dst)
    p = segsum(x, buckets, counts)
    h = mlp1(x, p, W1a, b1a.reshape(1, d), W1b, b1b.reshape(1, d))
    q = segsum(h, buckets, counts)
    out = mlp2(h, q, W2a, b2a.reshape(1, d), W2b, b2b.reshape(1, d))
    return out


# speedup vs baseline: 1.3841x; 1.3841x over previous
import functools

import jax
import jax.numpy as jnp
from jax import lax
from jax.experimental import pallas as pl
from jax.experimental.pallas import tpu as pltpu
from jax.experimental.pallas import tpu_sc as plsc

_NC = 2
_NS = 16
_NW = _NC * _NS
_CAP = 512
_G = 2000
_K = 128
_FLAT = 11520
_SHIFT = 14


def _rows_per_tile(n):
    return (-(-n // _NW) + 7) // 8 * 8


def _make_binner(n, e):
    tpe = e // _NW
    rng = _rows_per_tile(n)
    mesh = plsc.VectorSubcoreMesh(core_axis_name="c", subcore_axis_name="s")

    @functools.partial(
        pl.kernel,
        out_type=(
            jax.ShapeDtypeStruct((_NW, _NW, _CAP), jnp.int32),
            jax.ShapeDtypeStruct((_NW, _NW, 128), jnp.int32),
        ),
        mesh=mesh,
        compiler_params=pltpu.CompilerParams(needs_layout_passes=False),
        scratch_types=[
            pltpu.VMEM((_G,), jnp.int32),
            pltpu.VMEM((_G,), jnp.int32),
            pltpu.VMEM((_NW * _CAP,), jnp.int32),
            pltpu.VMEM((_NW, 128), jnp.int32),
        ],
    )
    def binner(src_hbm, dst_hbm, buck_hbm, cnt_hbm, srcb, dstb, buck, cntv):
        c = lax.axis_index("c")
        s = lax.axis_index("s")
        w = c * _NS + s
        ebase = w * tpe
        for r in range(_NW):
            cntv[r, pl.ds(0, 16)] = jnp.zeros((16,), jnp.int32)

        def _chunk(ch, carry):
            pltpu.sync_copy(src_hbm.at[pl.ds(ebase + ch * _G, _G)], srcb)
            pltpu.sync_copy(dst_hbm.at[pl.ds(ebase + ch * _G, _G)], dstb)

            lane = lax.iota(jnp.int32, 16)

            def _group(g, carry):
                sv = srcb[pl.ds(g * 16, 16)]
                dv = dstb[pl.ds(g * 16, 16)]
                pk = sv * (1 << _SHIFT) + dv
                sdv, spk = plsc.sort_key_val(dv, pk)
                for r in range(_NW):
                    m = (sdv >= r * rng) & (sdv < (r + 1) * rng)
                    cvec = cntv[r, pl.ds(0, 16)]
                    pos = cvec + lane - plsc.all_reduce_ffs(m)
                    plsc.store_scatter(buck, [pos + r * _CAP], spk, mask=m)
                    cntv[r, pl.ds(0, 16)] = (
                        cvec + plsc.all_reduce_population_count(m))
                return carry

            return lax.fori_loop(0, _G // 16, _group, carry)

        lax.fori_loop(0, tpe // _G, _chunk, 0)
        for r in range(_NW):
            pltpu.sync_copy(buck.at[pl.ds(r * _CAP, _CAP)], buck_hbm.at[r, w])
            pltpu.sync_copy(cntv.at[pl.ds(r, 1)], cnt_hbm.at[r, pl.ds(w, 1)])

    return binner


def _make_segsum(n, e, d):
    rng = _rows_per_tile(n)
    srows = rng + 16
    npad = _NW * rng
    mesh = plsc.VectorSubcoreMesh(core_axis_name="c", subcore_axis_name="s")

    @functools.partial(
        pl.kernel,
        out_type=jax.ShapeDtypeStruct((npad, d), jnp.float32),
        mesh=mesh,
        compiler_params=pltpu.CompilerParams(needs_layout_passes=False),
        scratch_types=[
            pltpu.VMEM((_NW, _CAP), jnp.int32),
            pltpu.VMEM((_NW, 128), jnp.int32),
            pltpu.VMEM((_FLAT,), jnp.int32),
            pltpu.VMEM((_K,), jnp.int32),
            pltpu.VMEM((_K,), jnp.int32),
            pltpu.VMEM((_K, d), jnp.float32),
            pltpu.VMEM((16, d), jnp.float32),
            pltpu.VMEM_SHARED((_NS * srows, d), jnp.float32),
            pltpu.SemaphoreType.DMA,
        ],
    )
    def segsum(h_hbm, buck_hbm, cnt_hbm, out_hbm, buckv, cntv, flat, srca,
               dsta, rowsa, zbuf, acc, sema):
        c = lax.axis_index("c")
        s = lax.axis_index("s")
        w = c * _NS + s
        sbase = s * srows

        def _zero_row(i, carry):
            for jj in range(d // 16):
                zbuf[i, pl.ds(jj * 16, 16)] = jnp.zeros((16,), jnp.float32)
            return carry

        lax.fori_loop(0, 16, _zero_row, 0)

        def _zero_acc(i, carry):
            pltpu.sync_copy(zbuf, acc.at[pl.ds(sbase + i * 16, 16)])
            return carry

        lax.fori_loop(0, srows // 16, _zero_acc, 0)

        pltpu.sync_copy(buck_hbm.at[w], buckv)
        pltpu.sync_copy(cnt_hbm.at[w], cntv)

        padv = jnp.full((16,), w * rng + rng, jnp.int32)
        lane = lax.iota(jnp.int32, 16)

        def _prefill(i, carry):
            flat[pl.ds(i * 16, 16)] = padv
            return carry

        lax.fori_loop(0, _FLAT // 16, _prefill, 0)

        tot = jnp.int32(0)
        for s2 in range(_NW):
            cnt = cntv[s2, pl.ds(0, 16)][0]

            def _cp(k, carry, s2=s2, cnt=cnt, tot=tot):
                vals = buckv[s2, pl.ds(k * 16, 16)]
                m = (k * 16 + lane) < cnt
                flat[pl.ds(tot + k * 16, 16)] = jnp.where(m, vals, padv)
                return carry

            lax.fori_loop(0, (cnt + 15) // 16, _cp, 0)
            tot = tot + cnt

        off = sbase - w * rng
        nch = (tot + _K - 1) // _K

        def _chunk(ch, carry):
            for k in range(_K // 16):
                p = flat[pl.ds(ch * _K + k * 16, 16)]
                srca[pl.ds(k * 16, 16)] = p >> _SHIFT
                dsta[pl.ds(k * 16, 16)] = (p & ((1 << _SHIFT) - 1)) + off
            pltpu.async_copy(h_hbm.at[srca], rowsa, sema).wait()
            pltpu.sync_copy(rowsa, acc.at[dsta], add=True)
            return carry

        lax.fori_loop(0, nch, _chunk, 0)

        pltpu.sync_copy(acc.at[pl.ds(sbase, rng)],
                        out_hbm.at[pl.ds(w * rng, rng)])

    return segsum


def _make_mlp(n, d, blk, final_relu):
    def body(x_ref, p_ref, wa_ref, ba_ref, wb_ref, bb_ref, o_ref):
        z = x_ref[...] + p_ref[...]
        t = jnp.dot(z, wa_ref[...], preferred_element_type=jnp.float32)
        t = jnp.maximum(t + ba_ref[...], 0.0)
        y = jnp.dot(t, wb_ref[...], preferred_element_type=jnp.float32)
        y = y + bb_ref[...]
        if final_relu:
            y = jnp.maximum(y, 0.0)
        o_ref[...] = y

    return pl.pallas_call(
        body,
        grid=(n // blk,),
        in_specs=[
            pl.BlockSpec((blk, d), lambda i: (i, 0)),
            pl.BlockSpec((blk, d), lambda i: (i, 0)),
            pl.BlockSpec((d, d), lambda i: (0, 0)),
            pl.BlockSpec((1, d), lambda i: (0, 0)),
            pl.BlockSpec((d, d), lambda i: (0, 0)),
            pl.BlockSpec((1, d), lambda i: (0, 0)),
        ],
        out_specs=pl.BlockSpec((blk, d), lambda i: (i, 0)),
        out_shape=jax.ShapeDtypeStruct((n, d), jnp.float32),
    )


def kernel(x, edge_index, W1a, b1a, W1b, b1b, W2a, b2a, W2b, b2b):
    n, d = x.shape
    e = edge_index.shape[1]
    src = edge_index[0]
    dst = edge_index[1]

    binner = _make_binner(n, e)
    segsum = _make_segsum(n, e, d)
    mlp1 = _make_mlp(n, d, 2000, True)
    mlp2 = _make_mlp(n, d, 2000, False)

    buckets, counts = binner(src, dst)
    p = segsum(x, buckets, counts)
    h = mlp1(x, p, W1a, b1a.reshape(1, d), W1b, b1b.reshape(1, d))
    q = segsum(h, buckets, counts)
    out = mlp2(h, q, W2a, b2a.reshape(1, d), W2b, b2b.reshape(1, d))
    return out


# speedup vs baseline: 1.4178x; 1.0244x over previous
import functools

import jax
import jax.numpy as jnp
from jax import lax
from jax.experimental import pallas as pl
from jax.experimental.pallas import tpu as pltpu
from jax.experimental.pallas import tpu_sc as plsc

_NC = 2
_NS = 16
_NW = _NC * _NS
_CAP = 512
_G = 2000
_K = 128
_FLAT = 11520
_SHIFT = 14


def _rows_per_tile(n):
    return (-(-n // _NW) + 7) // 8 * 8


def _make_binner(n, e):
    tpe = e // _NW
    rng = _rows_per_tile(n)
    mesh = plsc.VectorSubcoreMesh(core_axis_name="c", subcore_axis_name="s")

    @functools.partial(
        pl.kernel,
        out_type=(
            jax.ShapeDtypeStruct((_NW, _NW, _CAP), jnp.int32),
            jax.ShapeDtypeStruct((_NW, _NW, 128), jnp.int32),
        ),
        mesh=mesh,
        compiler_params=pltpu.CompilerParams(needs_layout_passes=False),
        scratch_types=[
            pltpu.VMEM((_G,), jnp.int32),
            pltpu.VMEM((_G,), jnp.int32),
            pltpu.VMEM((_NW * _CAP,), jnp.int32),
            pltpu.VMEM((_NW, 128), jnp.int32),
        ],
    )
    def binner(src_hbm, dst_hbm, buck_hbm, cnt_hbm, srcb, dstb, buck, cntv):
        c = lax.axis_index("c")
        s = lax.axis_index("s")
        w = c * _NS + s
        ebase = w * tpe
        for r in range(_NW):
            cntv[r, pl.ds(0, 16)] = jnp.zeros((16,), jnp.int32)

        def _chunk(ch, carry):
            pltpu.sync_copy(src_hbm.at[pl.ds(ebase + ch * _G, _G)], srcb)
            pltpu.sync_copy(dst_hbm.at[pl.ds(ebase + ch * _G, _G)], dstb)

            def _group(g, carry):
                sv = srcb[pl.ds(g * 16, 16)]
                dv = dstb[pl.ds(g * 16, 16)]
                pk = sv * (1 << _SHIFT) + dv
                for r in range(_NW):
                    m = (dv >= r * rng) & (dv < (r + 1) * rng)
                    cvec = cntv[r, pl.ds(0, 16)]
                    pos = cvec + plsc.cumsum(m.astype(jnp.int32)) - 1
                    plsc.store_scatter(buck, [pos + r * _CAP], pk, mask=m)
                    cntv[r, pl.ds(0, 16)] = (
                        cvec + plsc.all_reduce_population_count(m))
                return carry

            return lax.fori_loop(0, _G // 16, _group, carry)

        lax.fori_loop(0, tpe // _G, _chunk, 0)
        for r in range(_NW):
            pltpu.sync_copy(buck.at[pl.ds(r * _CAP, _CAP)], buck_hbm.at[r, w])
            pltpu.sync_copy(cntv.at[pl.ds(r, 1)], cnt_hbm.at[r, pl.ds(w, 1)])

    return binner


def _make_segsum(n, e, d):
    rng = _rows_per_tile(n)
    srows = rng + 16
    npad = _NW * rng
    mesh = plsc.VectorSubcoreMesh(core_axis_name="c", subcore_axis_name="s")

    @functools.partial(
        pl.kernel,
        out_type=jax.ShapeDtypeStruct((npad, d), jnp.float32),
        mesh=mesh,
        compiler_params=pltpu.CompilerParams(needs_layout_passes=False),
        scratch_types=[
            pltpu.VMEM((_NW, _CAP), jnp.int32),
            pltpu.VMEM((_NW, 128), jnp.int32),
            pltpu.VMEM((_FLAT,), jnp.int32),
            pltpu.VMEM((_K,), jnp.int32),
            pltpu.VMEM((_K,), jnp.int32),
            pltpu.VMEM((_K, d), jnp.float32),
            pltpu.VMEM((112, d), jnp.float32),
            pltpu.VMEM_SHARED((_NS * srows, d), jnp.float32),
            pltpu.SemaphoreType.DMA,
        ],
    )
    def segsum(h_hbm, buck_hbm, cnt_hbm, out_hbm, buckv, cntv, flat, srca,
               dsta, rowsa, zbuf, acc, sema):
        c = lax.axis_index("c")
        s = lax.axis_index("s")
        w = c * _NS + s
        sbase = s * srows

        def _zero_row(i, carry):
            for jj in range(d // 16):
                zbuf[i, pl.ds(jj * 16, 16)] = jnp.zeros((16,), jnp.float32)
            return carry

        lax.fori_loop(0, 112, _zero_row, 0)

        def _zero_acc(i, carry):
            pltpu.sync_copy(zbuf, acc.at[pl.ds(sbase + i * 112, 112)])
            return carry

        lax.fori_loop(0, srows // 112, _zero_acc, 0)

        pltpu.sync_copy(buck_hbm.at[w], buckv)
        pltpu.sync_copy(cnt_hbm.at[w], cntv)

        padv = jnp.full((16,), w * rng + rng, jnp.int32)
        lane = lax.iota(jnp.int32, 16)

        def _prefill(i, carry):
            flat[pl.ds(i * 16, 16)] = padv
            return carry

        lax.fori_loop(0, _FLAT // 16, _prefill, 0)

        tot = jnp.int32(0)
        for s2 in range(_NW):
            cnt = cntv[s2, pl.ds(0, 16)][0]

            def _cp(k, carry, s2=s2, cnt=cnt, tot=tot):
                vals = buckv[s2, pl.ds(k * 16, 16)]
                m = (k * 16 + lane) < cnt
                flat[pl.ds(tot + k * 16, 16)] = jnp.where(m, vals, padv)
                return carry

            lax.fori_loop(0, (cnt + 15) // 16, _cp, 0)
            tot = tot + cnt

        off = sbase - w * rng
        nch = (tot + _K - 1) // _K

        def _chunk(ch, carry):
            for k in range(_K // 16):
                p = flat[pl.ds(ch * _K + k * 16, 16)]
                srca[pl.ds(k * 16, 16)] = p >> _SHIFT
                dsta[pl.ds(k * 16, 16)] = (p & ((1 << _SHIFT) - 1)) + off
            pltpu.async_copy(h_hbm.at[srca], rowsa, sema).wait()
            pltpu.sync_copy(rowsa, acc.at[dsta], add=True)
            return carry

        lax.fori_loop(0, nch, _chunk, 0)

        pltpu.sync_copy(acc.at[pl.ds(sbase, rng)],
                        out_hbm.at[pl.ds(w * rng, rng)])

    return segsum


def _make_mlp(n, d, blk, final_relu):
    def body(x_ref, p_ref, wa_ref, ba_ref, wb_ref, bb_ref, o_ref):
        z = x_ref[...] + p_ref[...]
        t = jnp.dot(z, wa_ref[...], preferred_element_type=jnp.float32)
        t = jnp.maximum(t + ba_ref[...], 0.0)
        y = jnp.dot(t, wb_ref[...], preferred_element_type=jnp.float32)
        y = y + bb_ref[...]
        if final_relu:
            y = jnp.maximum(y, 0.0)
        o_ref[...] = y

    return pl.pallas_call(
        body,
        grid=(n // blk,),
        in_specs=[
            pl.BlockSpec((blk, d), lambda i: (i, 0)),
            pl.BlockSpec((blk, d), lambda i: (i, 0)),
            pl.BlockSpec((d, d), lambda i: (0, 0)),
            pl.BlockSpec((1, d), lambda i: (0, 0)),
            pl.BlockSpec((d, d), lambda i: (0, 0)),
            pl.BlockSpec((1, d), lambda i: (0, 0)),
        ],
        out_specs=pl.BlockSpec((blk, d), lambda i: (i, 0)),
        out_shape=jax.ShapeDtypeStruct((n, d), jnp.float32),
    )


def kernel(x, edge_index, W1a, b1a, W1b, b1b, W2a, b2a, W2b, b2b):
    n, d = x.shape
    e = edge_index.shape[1]
    src = edge_index[0]
    dst = edge_index[1]

    binner = _make_binner(n, e)
    segsum = _make_segsum(n, e, d)
    mlp1 = _make_mlp(n, d, 2000, True)
    mlp2 = _make_mlp(n, d, 2000, False)

    buckets, counts = binner(src, dst)
    p = segsum(x, buckets, counts)
    h = mlp1(x, p, W1a, b1a.reshape(1, d), W1b, b1b.reshape(1, d))
    q = segsum(h, buckets, counts)
    out = mlp2(h, q, W2a, b2a.reshape(1, d), W2b, b2b.reshape(1, d))
    return out
